# baseline (speedup 1.0000x reference)
import functools

import jax
import jax.numpy as jnp
from jax import lax
from jax.experimental import pallas as pl
from jax.experimental.pallas import tpu as pltpu
from jax.experimental.pallas import tpu_sc as plsc

NUM_WORKERS = 32
NBUF = 8


def _build_sc_gather(b: int, h: int, d: int):
    b_per_w = b // NUM_WORKERS
    mesh = plsc.VectorSubcoreMesh(core_axis_name="c", subcore_axis_name="s")

    @functools.partial(
        pl.kernel,
        mesh=mesh,
        out_type=jax.ShapeDtypeStruct((b, h, d), jnp.float32),
        compiler_params=pltpu.CompilerParams(use_tc_tiling_on_sc=True),
        scratch_types=[
            pltpu.VMEM((b_per_w, h), jnp.int32),
        ]
        + [pltpu.VMEM((h, d), jnp.float32) for _ in range(NBUF)]
        + [pltpu.SemaphoreType.DMA for _ in range(NBUF)],
    )
    def sc_gather(idx_hbm, table_hbm, out_hbm, idx_v, *bufs_and_sems):
        bufs = bufs_and_sems[:NBUF]
        sems = bufs_and_sems[NBUF:]
        wid = lax.axis_index("s") * 2 + lax.axis_index("c")
        base = wid * b_per_w
        pltpu.sync_copy(idx_hbm.at[wid], idx_v)
        for r in range(min(NBUF, b_per_w)):
            pltpu.async_copy(table_hbm.at[idx_v.at[r]], bufs[r], sems[r])
        for r in range(b_per_w):
            s = r % NBUF
            pltpu.make_async_copy(table_hbm.at[idx_v.at[r]], bufs[s], sems[s]).wait()
            pltpu.sync_copy(bufs[s], out_hbm.at[base + r])
            nxt = r + NBUF
            if nxt < b_per_w:
                pltpu.async_copy(table_hbm.at[idx_v.at[nxt]], bufs[s], sems[s])

    return sc_gather


def kernel(input_ids, word_embeddings):
    b, h = input_ids.shape
    v, d = word_embeddings.shape
    idx = input_ids.reshape(NUM_WORKERS, b // NUM_WORKERS, h).astype(jnp.int32)
    return _build_sc_gather(b, h, d)(idx, word_embeddings)


# speedup vs baseline: 1.7979x; 1.7979x over previous
import functools

import jax
import jax.numpy as jnp
from jax import lax
from jax.experimental import pallas as pl
from jax.experimental.pallas import tpu as pltpu
from jax.experimental.pallas import tpu_sc as plsc

NUM_WORKERS = 32
CHUNK = 64
NBUF = 10


def _build_sc_gather(b: int, h: int, d: int):
    b_per_w = b // NUM_WORKERS
    mesh = plsc.VectorSubcoreMesh(core_axis_name="c", subcore_axis_name="s")

    depth = 5
    halves = b_per_w // CHUNK
    n_chunks = h * halves
    assert n_chunks % NBUF == 0

    def out_slice(out_hbm, wid, blk, half):
        return out_hbm.at[blk * NUM_WORKERS + wid, pl.ds(half * CHUNK, CHUNK)]

    @functools.partial(
        pl.kernel,
        mesh=mesh,
        out_type=jax.ShapeDtypeStruct((h * NUM_WORKERS, b_per_w, d), jnp.float32),
        scratch_types=[
            pltpu.VMEM((n_chunks, CHUNK), jnp.int32),
        ]
        + [pltpu.VMEM((CHUNK, d), jnp.float32) for _ in range(NBUF)]
        + [pltpu.SemaphoreType.DMA for _ in range(2 * NBUF)],
    )
    def sc_gather(idx_hbm, table_hbm, out_hbm, idx_v, *bufs_and_sems):
        bufs = bufs_and_sems[:NBUF]
        gsems = bufs_and_sems[NBUF : 2 * NBUF]
        osems = bufs_and_sems[2 * NBUF :]
        wid = lax.axis_index("s") * 2 + lax.axis_index("c")
        pltpu.sync_copy(idx_hbm.at[wid], idx_v)
        for r in range(depth):
            pltpu.async_copy(table_hbm.at[idx_v.at[r]], bufs[r], gsems[r])

        @pl.loop(0, n_chunks // NBUF)
        def _group(g):
            for j in range(NBUF):
                c = g * NBUF + j
                blk = g * (NBUF // halves) + j // halves
                pltpu.make_async_copy(
                    table_hbm.at[idx_v.at[c]], bufs[j], gsems[j]
                ).wait()
                pltpu.async_copy(
                    bufs[j], out_slice(out_hbm, wid, blk, j % halves), osems[j]
                )
                nxt = c + depth
                jn = (j + depth) % NBUF
                nblk = (
                    g * (NBUF // halves) + (j + depth) // halves
                    if j + depth < NBUF
                    else (g + 1) * (NBUF // halves) + (j + depth - NBUF) // halves
                )
                pblk = nblk - NBUF // halves

                @pl.when(jnp.logical_and(nxt < n_chunks, nxt >= NBUF))
                def _wait_out():
                    pltpu.make_async_copy(
                        bufs[jn], out_slice(out_hbm, wid, pblk, jn % halves), osems[jn]
                    ).wait()

                @pl.when(nxt < n_chunks)
                def _issue_gather():
                    pltpu.async_copy(table_hbm.at[idx_v.at[nxt]], bufs[jn], gsems[jn])

        for r in range(n_chunks - NBUF, n_chunks):
            s = r % NBUF
            pltpu.make_async_copy(
                bufs[s],
                out_slice(out_hbm, wid, r // halves, r % halves),
                osems[s],
            ).wait()

    return sc_gather


def kernel(input_ids, word_embeddings):
    b, h = input_ids.shape
    v, d = word_embeddings.shape
    b_per_w = b // NUM_WORKERS
    idx = (
        input_ids.reshape(NUM_WORKERS, b_per_w, h)
        .transpose(0, 2, 1)
        .reshape(NUM_WORKERS, b_per_w * h // CHUNK, CHUNK)
        .astype(jnp.int32)
    )
    out = _build_sc_gather(b, h, d)(idx, word_embeddings)
    return out.reshape(h, b, d).transpose(1, 0, 2)


# speedup vs baseline: 1.8178x; 1.0110x over previous
import functools

import jax
import jax.numpy as jnp
from jax import lax
from jax.experimental import pallas as pl
from jax.experimental.pallas import tpu as pltpu
from jax.experimental.pallas import tpu_sc as plsc

NUM_WORKERS = 32
NBUF = 5


def _build_sc_gather(b: int, h: int, d: int):
    b_per_w = b // NUM_WORKERS
    mesh = plsc.VectorSubcoreMesh(core_axis_name="c", subcore_axis_name="s")

    depth = 3
    assert h % NBUF == 0

    @functools.partial(
        pl.kernel,
        mesh=mesh,
        out_type=jax.ShapeDtypeStruct((h * NUM_WORKERS, b_per_w, d), jnp.float32),
        scratch_types=[
            pltpu.VMEM((h, b_per_w), jnp.int32),
        ]
        + [pltpu.VMEM((b_per_w, d), jnp.float32) for _ in range(NBUF)]
        + [pltpu.SemaphoreType.DMA for _ in range(2 * NBUF)],
    )
    def sc_gather(idx_hbm, table_hbm, out_hbm, idx_v, *bufs_and_sems):
        bufs = bufs_and_sems[:NBUF]
        gsems = bufs_and_sems[NBUF : 2 * NBUF]
        osems = bufs_and_sems[2 * NBUF :]
        wid = lax.axis_index("s") * 2 + lax.axis_index("c")
        pltpu.sync_copy(idx_hbm.at[:, wid], idx_v)
        for r in range(min(depth, h)):
            pltpu.async_copy(table_hbm.at[idx_v.at[r]], bufs[r], gsems[r])

        @pl.loop(0, h // NBUF)
        def _group(g):
            for j in range(NBUF):
                c = g * NBUF + j
                pltpu.make_async_copy(
                    table_hbm.at[idx_v.at[c]], bufs[j], gsems[j]
                ).wait()
                pltpu.async_copy(bufs[j], out_hbm.at[c * NUM_WORKERS + wid], osems[j])
                nxt = c + depth
                s2 = (j + depth) % NBUF

                @pl.when(jnp.logical_and(nxt < h, nxt >= NBUF))
                def _wait_out():
                    pltpu.make_async_copy(
                        bufs[s2],
                        out_hbm.at[(nxt - NBUF) * NUM_WORKERS + wid],
                        osems[s2],
                    ).wait()

                @pl.when(nxt < h)
                def _issue_gather():
                    pltpu.async_copy(table_hbm.at[idx_v.at[nxt]], bufs[s2], gsems[s2])

        for r in range(max(0, h - NBUF), h):
            s = r % NBUF
            pltpu.make_async_copy(
                bufs[s], out_hbm.at[r * NUM_WORKERS + wid], osems[s]
            ).wait()

    return sc_gather


def kernel(input_ids, word_embeddings):
    b, h = input_ids.shape
    v, d = word_embeddings.shape
    b_per_w = b // NUM_WORKERS
    idx = input_ids.T.reshape(h, NUM_WORKERS, b_per_w).astype(jnp.int32)
    out = _build_sc_gather(b, h, d)(idx, word_embeddings)
    return out.reshape(h, b, d).transpose(1, 0, 2)
